# baseline (speedup 1.0000x reference)
import functools

import jax
import jax.numpy as jnp
from jax import lax
from jax.experimental import pallas as pl
from jax.experimental.pallas import tpu as pltpu
from jax.experimental.pallas import tpu_sc as plsc

NUM_GLYPHS = 5976
LUT_PAD = 6016
D = 64
NC, NS = 2, 16
NW = NC * NS
BB = 128
NTB = 1024 // BB
SPW = 200 // (NW // NTB)


def _make_lookup():
    mesh = plsc.VectorSubcoreMesh(
        core_axis_name="c", subcore_axis_name="s",
        num_cores=NC, num_subcores=NS)

    @functools.partial(
        pl.kernel, mesh=mesh,
        compiler_params=pltpu.CompilerParams(
            needs_layout_passes=False, use_tc_tiling_on_sc=False),
        out_type=jax.ShapeDtypeStruct((200, 8, NTB, 8, BB), jnp.float32),
        scratch_types=[
            pltpu.VMEM((BB, 200), jnp.int32),
            pltpu.VMEM((LUT_PAD,), jnp.int32),
            pltpu.VMEM((LUT_PAD,), jnp.int32),
            pltpu.VMEM((13, D), jnp.float32),
            pltpu.VMEM((2, BB), jnp.int32),
            pltpu.VMEM((2, BB), jnp.int32),
            pltpu.VMEM((2, BB, D), jnp.float32),
            pltpu.VMEM((2, 8, 8, BB), jnp.float32),
            pltpu.SemaphoreType.DMA,
            pltpu.SemaphoreType.DMA,
            pltpu.SemaphoreType.DMA,
            pltpu.SemaphoreType.DMA,
        ],
    )
    def lookup(ent_hbm, grp_hbm, elut_hbm, glut_hbm, gl_hbm, out_hbm,
               gl_v, elut_v, glut_v, grp_v, idx_v, gg_v, rows_v, tile_v,
               gsem0, gsem1, wsem0, wsem1):
        wid = lax.axis_index("s") * NC + lax.axis_index("c")
        tb = wid % NTB
        s0 = (wid // NTB) * SPW
        pltpu.sync_copy(gl_hbm.at[pl.ds(tb * BB, BB), :], gl_v)
        pltpu.sync_copy(elut_hbm, elut_v)
        pltpu.sync_copy(glut_hbm, glut_v)
        pltpu.sync_copy(grp_hbm, grp_v)
        lanes = lax.iota(jnp.int32, 16)
        gsems = (gsem0, gsem1)
        wsems = (wsem0, wsem1)

        def indices(s, slot):
            s16 = jnp.full((16,), s, jnp.int32)
            for t in range(BB // 16):
                b16 = lanes + (t * 16)
                g16 = plsc.load_gather(gl_v, [b16, s16])
                idx_v[slot, pl.ds(t * 16, 16)] = plsc.load_gather(
                    elut_v, [g16])
                gg_v[slot, pl.ds(t * 16, 16)] = plsc.load_gather(
                    glut_v, [g16])

        def gather_start(slot):
            pltpu.async_copy(ent_hbm.at[idx_v.at[slot]], rows_v.at[slot],
                             gsems[slot])

        def gather_wait(slot):
            pltpu.make_async_copy(ent_hbm.at[idx_v.at[slot]],
                                  rows_v.at[slot], gsems[slot]).wait()

        def transpose_add(slot):
            for q in range(BB // 16):
                lb16 = lanes + (q * 16)
                gg16 = gg_v[slot, pl.ds(q * 16, 16)]
                for d in range(D):
                    d16 = jnp.full((16,), d, jnp.int32)
                    val = plsc.load_gather(rows_v.at[slot], [lb16, d16])
                    val = val + plsc.load_gather(grp_v, [gg16, d16])
                    tile_v[slot, d // 8, d % 8, pl.ds(q * 16, 16)] = val

        def write_start(slot, s):
            pltpu.async_copy(tile_v.at[slot], out_hbm.at[s, :, tb],
                             wsems[slot])

        def write_wait(slot, s):
            pltpu.make_async_copy(tile_v.at[slot], out_hbm.at[s, :, tb],
                                  wsems[slot]).wait()

        indices(s0, 0)
        gather_start(0)

        def step(u, carry):
            s_a = s0 + u * 2
            s_b = s_a + 1
            indices(s_b, 1)
            gather_start(1)
            gather_wait(0)

            @pl.when(u > 0)
            def _():
                write_wait(0, s_a - 2)

            transpose_add(0)
            write_start(0, s_a)

            @pl.when(u < SPW // 2 - 1)
            def _():
                indices(s_a + 2, 0)
                gather_start(0)

            gather_wait(1)

            @pl.when(u > 0)
            def _():
                write_wait(1, s_b - 2)

            transpose_add(1)
            write_start(1, s_b)
            return carry

        lax.fori_loop(0, SPW // 2, step, 0)
        write_wait(0, s0 + SPW - 2)
        write_wait(1, s0 + SPW - 1)

    return lookup


_lookup = _make_lookup()


def kernel(glyphs, entity_lut, group_lut, entity_table, group_table):
    gl = glyphs.astype(jnp.int32)
    elut = jnp.pad(entity_lut.astype(jnp.int32), (0, LUT_PAD - NUM_GLYPHS))
    glut = jnp.pad(group_lut.astype(jnp.int32), (0, LUT_PAD - NUM_GLYPHS))
    x5 = _lookup(entity_table, group_table, elut, glut, gl)
    y = jnp.transpose(x5, (2, 4, 0, 1, 3))
    return y.reshape(1024, 200, D)


# speedup vs baseline: 2.9587x; 2.9587x over previous
import functools

import jax
import jax.numpy as jnp
from jax import lax
from jax.experimental import pallas as pl
from jax.experimental.pallas import tpu as pltpu
from jax.experimental.pallas import tpu_sc as plsc

NUM_GLYPHS = 5976
LUT_PAD = 6016
ENT_PAD = 2048
NGRP = 13
D = 64
NC, NS = 2, 16
NW = NC * NS
CH = 128
N_TOTAL = 1024 * 200
NPW = N_TOTAL // NW
NCH = NPW // CH


def _prep_body(ent_ref, grp_ref, out_ref):
    out_ref[...] = ent_ref[...] + grp_ref[0]


_prep = pl.pallas_call(
    _prep_body,
    grid=(NGRP,),
    in_specs=[
        pl.BlockSpec((ENT_PAD, D), lambda j: (0, 0)),
        pl.BlockSpec((1, 1, D), lambda j: (j, 0, 0)),
    ],
    out_specs=pl.BlockSpec((ENT_PAD, D), lambda j: (j, 0)),
    out_shape=jax.ShapeDtypeStruct((NGRP * ENT_PAD, D), jnp.float32),
)


def _make_lookup():
    mesh = plsc.VectorSubcoreMesh(
        core_axis_name="c", subcore_axis_name="s",
        num_cores=NC, num_subcores=NS)

    @functools.partial(
        pl.kernel, mesh=mesh,
        compiler_params=pltpu.CompilerParams(
            needs_layout_passes=False, use_tc_tiling_on_sc=False),
        out_type=jax.ShapeDtypeStruct((N_TOTAL // CH, CH, D), jnp.float32),
        scratch_types=[
            pltpu.VMEM((NPW,), jnp.int32),
            pltpu.VMEM((LUT_PAD,), jnp.int32),
            pltpu.VMEM((LUT_PAD,), jnp.int32),
            pltpu.VMEM((4, CH), jnp.int32),
            pltpu.VMEM((4, CH, D), jnp.float32),
            pltpu.SemaphoreType.DMA,
            pltpu.SemaphoreType.DMA,
            pltpu.SemaphoreType.DMA,
            pltpu.SemaphoreType.DMA,
            pltpu.SemaphoreType.DMA,
            pltpu.SemaphoreType.DMA,
            pltpu.SemaphoreType.DMA,
            pltpu.SemaphoreType.DMA,
        ],
    )
    def lookup(ct_hbm, elut_hbm, glut_hbm, gl_hbm, out_hbm,
               gl_v, elut_v, glut_v, idx_v, rows_v,
               gsem0, gsem1, gsem2, gsem3, wsem0, wsem1, wsem2, wsem3):
        wid = lax.axis_index("s") * NC + lax.axis_index("c")
        base = pl.multiple_of(wid * NPW, NPW)
        kbase = pl.multiple_of(wid * NCH, NCH)
        pltpu.sync_copy(gl_hbm.at[pl.ds(base, NPW)], gl_v)
        pltpu.sync_copy(elut_hbm, elut_v)
        pltpu.sync_copy(glut_hbm, glut_v)
        gsems = (gsem0, gsem1, gsem2, gsem3)
        wsems = (wsem0, wsem1, wsem2, wsem3)

        def indices(j, slot):
            off = pl.multiple_of(j * CH, CH)
            for t in range(CH // 16):
                g = gl_v[pl.ds(off + t * 16, 16)]
                ge = plsc.load_gather(elut_v, [g])
                gg = plsc.load_gather(glut_v, [g])
                idx_v[slot, pl.ds(t * 16, 16)] = gg * ENT_PAD + ge

        def gather_start(slot):
            pltpu.async_copy(ct_hbm.at[idx_v.at[slot]], rows_v.at[slot],
                             gsems[slot])

        def gather_wait(slot):
            pltpu.make_async_copy(ct_hbm.at[idx_v.at[slot]],
                                  rows_v.at[slot], gsems[slot]).wait()

        def write_start(slot, j):
            pltpu.async_copy(rows_v.at[slot], out_hbm.at[kbase + j],
                             wsems[slot])

        def write_wait(slot, j):
            pltpu.make_async_copy(rows_v.at[slot], out_hbm.at[kbase + j],
                                  wsems[slot]).wait()

        indices(0, 0)
        gather_start(0)
        indices(1, 1)
        gather_start(1)

        def step(u, carry):
            for k in range(4):
                j = u * 4 + k
                jn = j + 2
                sn = (k + 2) % 4
                indices(jn, sn)
                if k < 2:
                    @pl.when(u > 0)
                    def _():
                        write_wait(sn, jn - 4)
                else:
                    write_wait(sn, jn - 4)
                gather_start(sn)
                gather_wait(k)
                write_start(k, j)
            return carry

        lax.fori_loop(0, (NCH - 2) // 4, step, 0)
        gather_wait(0)
        write_wait(2, NCH - 4)
        write_start(0, NCH - 2)
        gather_wait(1)
        write_wait(3, NCH - 3)
        write_start(1, NCH - 2 + 1)
        write_wait(0, NCH - 2)
        write_wait(1, NCH - 1)

    return lookup


_lookup = _make_lookup()


def kernel(glyphs, entity_lut, group_lut, entity_table, group_table):
    b, s = glyphs.shape
    gl = glyphs.astype(jnp.int32).reshape(b * s)
    elut = jnp.pad(entity_lut.astype(jnp.int32), (0, LUT_PAD - NUM_GLYPHS))
    glut = jnp.pad(group_lut.astype(jnp.int32), (0, LUT_PAD - NUM_GLYPHS))
    ent_p = jnp.pad(entity_table,
                    ((0, ENT_PAD - entity_table.shape[0]), (0, 0)))
    grp3 = group_table.reshape(NGRP, 1, D)
    ctable = _prep(ent_p, grp3)
    out = _lookup(ctable, elut, glut, gl)
    return out.reshape(b, s, D)


# speedup vs baseline: 3.1420x; 1.0619x over previous
import functools

import jax
import jax.numpy as jnp
from jax import lax
from jax.experimental import pallas as pl
from jax.experimental.pallas import tpu as pltpu
from jax.experimental.pallas import tpu_sc as plsc

NUM_GLYPHS = 5976
LUT_PAD = 6016
ENT_PAD = 2048
NGRP = 13
D = 64
NC, NS = 2, 16
NW = NC * NS
CH = 128
N_TOTAL = 1024 * 200
NPW = N_TOTAL // NW
NCH = NPW // CH


def _prep_body(ent_ref, grp_ref, out_ref):
    out_ref[...] = ent_ref[...] + grp_ref[0]


_prep = pl.pallas_call(
    _prep_body,
    grid=(NGRP,),
    in_specs=[
        pl.BlockSpec((ENT_PAD // 2, 2 * D), lambda j: (0, 0)),
        pl.BlockSpec((1, 1, 2 * D), lambda j: (j, 0, 0)),
    ],
    out_specs=pl.BlockSpec((ENT_PAD // 2, 2 * D), lambda j: (j, 0)),
    out_shape=jax.ShapeDtypeStruct((NGRP * ENT_PAD // 2, 2 * D),
                                   jnp.float32),
)


def _make_lookup():
    mesh = plsc.VectorSubcoreMesh(
        core_axis_name="c", subcore_axis_name="s",
        num_cores=NC, num_subcores=NS)

    @functools.partial(
        pl.kernel, mesh=mesh,
        compiler_params=pltpu.CompilerParams(
            needs_layout_passes=False, use_tc_tiling_on_sc=False),
        out_type=jax.ShapeDtypeStruct((N_TOTAL // CH, CH, D), jnp.float32),
        scratch_types=[
            pltpu.VMEM((NPW,), jnp.int32),
            pltpu.VMEM((LUT_PAD,), jnp.int32),
            pltpu.VMEM((LUT_PAD,), jnp.int32),
            pltpu.VMEM((4, CH), jnp.int32),
            pltpu.VMEM((4, CH, D), jnp.float32),
            pltpu.SemaphoreType.DMA,
            pltpu.SemaphoreType.DMA,
            pltpu.SemaphoreType.DMA,
            pltpu.SemaphoreType.DMA,
            pltpu.SemaphoreType.DMA,
            pltpu.SemaphoreType.DMA,
            pltpu.SemaphoreType.DMA,
            pltpu.SemaphoreType.DMA,
        ],
    )
    def lookup(ct_hbm, elut_hbm, glut_hbm, gl_hbm, out_hbm,
               gl_v, elut_v, glut_v, idx_v, rows_v,
               gsem0, gsem1, gsem2, gsem3, wsem0, wsem1, wsem2, wsem3):
        wid = lax.axis_index("s") * NC + lax.axis_index("c")
        base = pl.multiple_of(wid * NPW, NPW)
        kbase = pl.multiple_of(wid * NCH, NCH)
        pltpu.sync_copy(gl_hbm.at[pl.ds(base, NPW)], gl_v)
        pltpu.sync_copy(elut_hbm, elut_v)
        pltpu.sync_copy(glut_hbm, glut_v)
        gsems = (gsem0, gsem1, gsem2, gsem3)
        wsems = (wsem0, wsem1, wsem2, wsem3)

        def indices(j, slot):
            off = pl.multiple_of(j * CH, CH)
            for t in range(CH // 16):
                g = gl_v[pl.ds(off + t * 16, 16)]
                ge = plsc.load_gather(elut_v, [g])
                gg = plsc.load_gather(glut_v, [g])
                idx_v[slot, pl.ds(t * 16, 16)] = gg * ENT_PAD + ge

        def gather_start(slot):
            pltpu.async_copy(ct_hbm.at[idx_v.at[slot]], rows_v.at[slot],
                             gsems[slot])

        def gather_wait(slot):
            pltpu.make_async_copy(ct_hbm.at[idx_v.at[slot]],
                                  rows_v.at[slot], gsems[slot]).wait()

        def write_start(slot, j):
            pltpu.async_copy(rows_v.at[slot], out_hbm.at[kbase + j],
                             wsems[slot])

        def write_wait(slot, j):
            pltpu.make_async_copy(rows_v.at[slot], out_hbm.at[kbase + j],
                                  wsems[slot]).wait()

        indices(0, 0)
        gather_start(0)
        indices(1, 1)
        gather_start(1)

        def step(u, carry):
            for k in range(4):
                j = u * 4 + k
                jn = j + 2
                sn = (k + 2) % 4
                indices(jn, sn)
                if k < 2:
                    @pl.when(u > 0)
                    def _():
                        write_wait(sn, jn - 4)
                else:
                    write_wait(sn, jn - 4)
                gather_start(sn)
                gather_wait(k)
                write_start(k, j)
            return carry

        lax.fori_loop(0, (NCH - 2) // 4, step, 0)
        gather_wait(0)
        write_wait(2, NCH - 4)
        write_start(0, NCH - 2)
        gather_wait(1)
        write_wait(3, NCH - 3)
        write_start(1, NCH - 2 + 1)
        write_wait(0, NCH - 2)
        write_wait(1, NCH - 1)

    return lookup


_lookup = _make_lookup()


def kernel(glyphs, entity_lut, group_lut, entity_table, group_table):
    b, s = glyphs.shape
    gl = glyphs.astype(jnp.int32).reshape(b * s)
    elut = jnp.pad(entity_lut.astype(jnp.int32), (0, LUT_PAD - NUM_GLYPHS))
    glut = jnp.pad(group_lut.astype(jnp.int32), (0, LUT_PAD - NUM_GLYPHS))
    ent_p = jnp.pad(entity_table,
                    ((0, ENT_PAD - entity_table.shape[0]), (0, 0)))
    ent_p2 = ent_p.reshape(ENT_PAD // 2, 2 * D)
    grp3 = jnp.concatenate([group_table, group_table],
                           axis=1).reshape(NGRP, 1, 2 * D)
    ctable = _prep(ent_p2, grp3).reshape(NGRP * ENT_PAD, D)
    out = _lookup(ctable, elut, glut, gl)
    return out.reshape(b, s, D)
